# baseline (speedup 1.0000x reference)
import functools

import jax
import jax.numpy as jnp
from jax import lax
from jax.experimental import pallas as pl
from jax.experimental.pallas import tpu as pltpu
from jax.experimental.pallas import tpu_sc as plsc

D = 64
B = 16384
NC = 2
NS = 16
L = 16
NW = NC * NS
BPW = B // NW

_mesh = plsc.VectorSubcoreMesh(core_axis_name="c", subcore_axis_name="s")


@functools.partial(
    pl.kernel,
    out_type=jax.ShapeDtypeStruct((B,), jnp.float32),
    mesh=_mesh,
    compiler_params=pltpu.CompilerParams(
        needs_layout_passes=False, use_tc_tiling_on_sc=False),
    scratch_types=[
        pltpu.VMEM((BPW,), jnp.int32),
        pltpu.VMEM((BPW,), jnp.int32),
        pltpu.VMEM((BPW, D), jnp.bfloat16),
        pltpu.VMEM((BPW, D), jnp.bfloat16),
        pltpu.VMEM((BPW * L,), jnp.float32),
        pltpu.VMEM((BPW,), jnp.float32),
        pltpu.SemaphoreType.DMA,
        pltpu.SemaphoreType.DMA,
    ],
)
def _sc_dots(target_hbm, context_hbm, w_in_hbm, w_out_hbm, out_hbm,
             idx_t, idx_c, rows_t, rows_c, parts, dots, sem_t, sem_c):
    wid = lax.axis_index("s") * NC + lax.axis_index("c")
    base = wid * BPW

    pltpu.sync_copy(target_hbm.at[pl.ds(base, BPW)], idx_t)
    pltpu.sync_copy(context_hbm.at[pl.ds(base, BPW)], idx_c)
    cp_t = pltpu.async_copy(w_in_hbm.at[idx_t], rows_t, sem_t)
    cp_c = pltpu.async_copy(w_out_hbm.at[idx_c], rows_c, sem_c)
    cp_t.wait()
    cp_c.wait()

    lanes = lax.iota(jnp.int32, L)

    def prod_body(r, carry):
        part = jnp.zeros((L,), jnp.float32)
        for k in range(D // (2 * L)):
            tlo, thi = plsc.unpack(
                rows_t[r, pl.ds(k * 2 * L, 2 * L)],
                format=plsc.PackFormat.INTERLEAVED)
            clo, chi = plsc.unpack(
                rows_c[r, pl.ds(k * 2 * L, 2 * L)],
                format=plsc.PackFormat.INTERLEAVED)
            part = part + tlo * clo + thi * chi
        parts[pl.ds(r * L, L)] = part
        return carry

    lax.fori_loop(0, BPW, prod_body, 0)

    def sum_body(g, carry):
        row_base = (g * L + lanes) * L
        acc = plsc.load_gather(parts, [row_base])
        for j in range(1, L):
            acc = acc + plsc.load_gather(parts, [row_base + j])
        dots[pl.ds(g * L, L)] = acc
        return carry

    lax.fori_loop(0, BPW // L, sum_body, 0)
    pltpu.sync_copy(dots, out_hbm.at[pl.ds(base, BPW)])


def _bce_body(z_ref, y_ref, out_ref):
    z = z_ref[...]
    y = y_ref[...].astype(jnp.float32)
    p = jax.nn.sigmoid(z)
    eps = 1e-12
    p = jnp.clip(p, eps, 1.0 - eps)
    loss = y * jnp.log(p) + (1.0 - y) * jnp.log(1.0 - p)
    out_ref[0, 0] = -jnp.sum(loss) / B


def kernel(target, context, labels, W_in, W_out):
    wb_in = W_in.astype(jnp.bfloat16)
    wb_out = W_out.astype(jnp.bfloat16)
    dots = _sc_dots(target, context, wb_in, wb_out)
    loss = pl.pallas_call(
        _bce_body,
        out_shape=jax.ShapeDtypeStruct((1, 1), jnp.float32),
        out_specs=pl.BlockSpec(memory_space=pltpu.SMEM),
    )(dots.reshape(128, 128), labels.reshape(128, 128))
    return loss[0, 0]


# speedup vs baseline: 1.5291x; 1.5291x over previous
import functools

import jax
import jax.numpy as jnp
from jax import lax
from jax.experimental import pallas as pl
from jax.experimental.pallas import tpu as pltpu
from jax.experimental.pallas import tpu_sc as plsc

V = 1000000
D = 64
B = 16384
NC = 2
NS = 16
L = 16
NW = NC * NS
BPW = B // NW
CH = 256

TCV = 2048
TPB = TCV // 2
TGRID = (V + TCV - 1) // TCV

_mesh = plsc.VectorSubcoreMesh(core_axis_name="c", subcore_axis_name="s")


def _pack_body(wt_ref, g_ref):
    x = wt_ref[...]
    y = x.T
    g_ref[...] = jnp.concatenate([y[:TPB], y[TPB:]], axis=1)


def _pack(wt):
    return pl.pallas_call(
        _pack_body,
        grid=(TGRID,),
        in_specs=[pl.BlockSpec((D, TCV), lambda i: (0, i))],
        out_specs=pl.BlockSpec((TPB, 2 * D), lambda i: (i, 0)),
        out_shape=jax.ShapeDtypeStruct((TGRID * TPB, 2 * D), jnp.float32),
    )(wt)


@functools.partial(
    pl.kernel,
    out_type=jax.ShapeDtypeStruct((B,), jnp.float32),
    mesh=_mesh,
    compiler_params=pltpu.CompilerParams(needs_layout_passes=False),
    scratch_types=[
        pltpu.VMEM((BPW,), jnp.int32),
        pltpu.VMEM((BPW,), jnp.int32),
        pltpu.VMEM((BPW,), jnp.int32),
        pltpu.VMEM((BPW,), jnp.int32),
        pltpu.VMEM((CH, 2 * D), jnp.float32),
        pltpu.VMEM((CH, 2 * D), jnp.float32),
        pltpu.VMEM((BPW,), jnp.float32),
        pltpu.SemaphoreType.DMA,
        pltpu.SemaphoreType.DMA,
    ],
)
def _sc_dots(target_hbm, context_hbm, g_in_hbm, g_out_hbm, out_hbm,
             idx_t, idx_c, slab_t, slab_c, buf_t, buf_c, dots,
             sem_t, sem_c):
    wid = lax.axis_index("s") * NC + lax.axis_index("c")
    base = wid * BPW

    pltpu.sync_copy(target_hbm.at[pl.ds(base, BPW)], idx_t)
    pltpu.sync_copy(context_hbm.at[pl.ds(base, BPW)], idx_c)

    def slab_body(g, carry):
        vt = idx_t[pl.ds(g * L, L)]
        vc = idx_c[pl.ds(g * L, L)]
        slab_t[pl.ds(g * L, L)] = ((vt >> 11) << 10) + (vt & 1023)
        slab_c[pl.ds(g * L, L)] = ((vc >> 11) << 10) + (vc & 1023)
        return carry

    lax.fori_loop(0, BPW // L, slab_body, 0)

    lanes = lax.iota(jnp.int32, L)

    for h in range(BPW // CH):
        cp_t = pltpu.async_copy(
            g_in_hbm.at[slab_t.at[pl.ds(h * CH, CH)]], buf_t, sem_t)
        cp_c = pltpu.async_copy(
            g_out_hbm.at[slab_c.at[pl.ds(h * CH, CH)]], buf_c, sem_c)
        cp_t.wait()
        cp_c.wait()

        def grp_body(gg, carry):
            s0 = h * CH + gg * L
            vt = idx_t[pl.ds(s0, L)]
            vc = idx_c[pl.ds(s0, L)]
            lid = gg * L + lanes
            ot = ((vt >> 10) & 1) * D
            oc = ((vc >> 10) & 1) * D
            acc = jnp.zeros((L,), jnp.float32)
            for d in range(D):
                tv = plsc.load_gather(buf_t, [lid, ot + d])
                cv = plsc.load_gather(buf_c, [lid, oc + d])
                acc = acc + tv * cv
            dots[pl.ds(s0, L)] = acc
            return carry

        lax.fori_loop(0, CH // L, grp_body, 0)

    pltpu.sync_copy(dots, out_hbm.at[pl.ds(base, BPW)])


def _bce_body(z_ref, y_ref, out_ref):
    z = z_ref[...]
    y = y_ref[...].astype(jnp.float32)
    p = jax.nn.sigmoid(z)
    eps = 1e-12
    p = jnp.clip(p, eps, 1.0 - eps)
    loss = y * jnp.log(p) + (1.0 - y) * jnp.log(1.0 - p)
    out_ref[0, 0] = -jnp.sum(loss) / B


def kernel(target, context, labels, W_in, W_out):
    g_in = _pack(W_in.T)
    g_out = _pack(W_out.T)
    dots = _sc_dots(target, context, g_in, g_out)
    loss = pl.pallas_call(
        _bce_body,
        out_shape=jax.ShapeDtypeStruct((1, 1), jnp.float32),
        out_specs=pl.BlockSpec(memory_space=pltpu.SMEM),
    )(dots.reshape(128, 128), labels.reshape(128, 128))
    return loss[0, 0]


# speedup vs baseline: 3.9352x; 2.5736x over previous
import functools

import jax
import jax.numpy as jnp
from jax import lax
from jax.experimental import pallas as pl
from jax.experimental.pallas import tpu as pltpu
from jax.experimental.pallas import tpu_sc as plsc

V = 1000000
D = 64
B = 16384
NC = 2
NS = 16
L = 16
NW = NC * NS
BPW = B // NW
CH = 256

TCV = 8192
Q = TCV // 4
TGRID = (V + TCV - 1) // TCV

_mesh = plsc.VectorSubcoreMesh(core_axis_name="c", subcore_axis_name="s")


def _pack_body(wt_ref, g_ref):
    x = wt_ref[...]
    b = lax.bitcast_convert_type(x, jnp.int32)
    r = (b + 0x7FFF + ((b >> 16) & 1)) >> 16
    lo = r[:D // 2] & 0xFFFF
    hi = r[D // 2:] << 16
    xp = hi | lo
    z = jnp.concatenate(
        [xp[:, q * Q:(q + 1) * Q] for q in range(4)], axis=0)
    g_ref[...] = z.T


def _pack(wt):
    return pl.pallas_call(
        _pack_body,
        grid=(TGRID,),
        in_specs=[pl.BlockSpec((D, TCV), lambda i: (0, i))],
        out_specs=pl.BlockSpec((Q, 2 * D), lambda i: (i, 0)),
        out_shape=jax.ShapeDtypeStruct((TGRID * Q, 2 * D), jnp.int32),
    )(wt)


@functools.partial(
    pl.kernel,
    out_type=jax.ShapeDtypeStruct((B,), jnp.float32),
    mesh=_mesh,
    compiler_params=pltpu.CompilerParams(needs_layout_passes=False),
    scratch_types=[
        pltpu.VMEM((BPW,), jnp.int32),
        pltpu.VMEM((BPW,), jnp.int32),
        pltpu.VMEM((BPW,), jnp.int32),
        pltpu.VMEM((BPW,), jnp.int32),
        pltpu.VMEM((CH, 2 * D), jnp.int32),
        pltpu.VMEM((CH, 2 * D), jnp.int32),
        pltpu.VMEM((BPW,), jnp.float32),
        pltpu.SemaphoreType.DMA,
        pltpu.SemaphoreType.DMA,
    ],
)
def _sc_dots(target_hbm, context_hbm, g_in_hbm, g_out_hbm, out_hbm,
             idx_t, idx_c, slab_t, slab_c, buf_t, buf_c, dots,
             sem_t, sem_c):
    wid = lax.axis_index("s") * NC + lax.axis_index("c")
    base = wid * BPW

    pltpu.sync_copy(target_hbm.at[pl.ds(base, BPW)], idx_t)
    pltpu.sync_copy(context_hbm.at[pl.ds(base, BPW)], idx_c)

    def slab_body(g, carry):
        vt = idx_t[pl.ds(g * L, L)]
        vc = idx_c[pl.ds(g * L, L)]
        slab_t[pl.ds(g * L, L)] = ((vt >> 13) << 11) + (vt & 2047)
        slab_c[pl.ds(g * L, L)] = ((vc >> 13) << 11) + (vc & 2047)
        return carry

    lax.fori_loop(0, BPW // L, slab_body, 0)

    lanes = lax.iota(jnp.int32, L)

    for h in range(BPW // CH):
        cp_t = pltpu.async_copy(
            g_in_hbm.at[slab_t.at[pl.ds(h * CH, CH)]], buf_t, sem_t)
        cp_c = pltpu.async_copy(
            g_out_hbm.at[slab_c.at[pl.ds(h * CH, CH)]], buf_c, sem_c)
        cp_t.wait()
        cp_c.wait()

        def grp_body(gg, carry):
            s0 = h * CH + gg * L
            vt = idx_t[pl.ds(s0, L)]
            vc = idx_c[pl.ds(s0, L)]
            lid = gg * L + lanes
            ot = ((vt >> 11) & 3) * 32
            oc = ((vc >> 11) & 3) * 32
            acc = jnp.zeros((L,), jnp.float32)
            for dp in range(D // 2):
                tw = plsc.load_gather(buf_t, [lid, ot + dp])
                cw = plsc.load_gather(buf_c, [lid, oc + dp])
                tlo, thi = plsc.unpack(
                    plsc.bitcast(tw, jnp.bfloat16),
                    format=plsc.PackFormat.INTERLEAVED)
                clo, chi = plsc.unpack(
                    plsc.bitcast(cw, jnp.bfloat16),
                    format=plsc.PackFormat.INTERLEAVED)
                acc = acc + tlo * clo + thi * chi
            dots[pl.ds(s0, L)] = acc
            return carry

        lax.fori_loop(0, CH // L, grp_body, 0)

    pltpu.sync_copy(dots, out_hbm.at[pl.ds(base, BPW)])


def _bce_body(z_ref, y_ref, out_ref):
    z = z_ref[...]
    y = y_ref[...].astype(jnp.float32)
    p = jax.nn.sigmoid(z)
    eps = 1e-12
    p = jnp.clip(p, eps, 1.0 - eps)
    loss = y * jnp.log(p) + (1.0 - y) * jnp.log(1.0 - p)
    out_ref[0, 0] = -jnp.sum(loss) / B


def kernel(target, context, labels, W_in, W_out):
    g_in = _pack(W_in.T)
    g_out = _pack(W_out.T)
    dots = _sc_dots(target, context, g_in, g_out)
    loss = pl.pallas_call(
        _bce_body,
        out_shape=jax.ShapeDtypeStruct((1, 1), jnp.float32),
        out_specs=pl.BlockSpec(memory_space=pltpu.SMEM),
    )(dots.reshape(128, 128), labels.reshape(128, 128))
    return loss[0, 0]


# speedup vs baseline: 4.8426x; 1.2306x over previous
import functools

import jax
import jax.numpy as jnp
from jax import lax
from jax.experimental import pallas as pl
from jax.experimental.pallas import tpu as pltpu
from jax.experimental.pallas import tpu_sc as plsc

V = 1000000
D = 64
B = 16384
NC = 2
NS = 16
L = 16
NW = NC * NS
BPW = B // NW
CH = 256

TCV = 16384
Q = TCV // 4
TGRID = (V + TCV - 1) // TCV

_mesh = plsc.VectorSubcoreMesh(core_axis_name="c", subcore_axis_name="s")


def _pack_body(wt_ref, g_ref):
    x = wt_ref[...]
    b = lax.bitcast_convert_type(x, jnp.int32)
    r = (b + 0x7FFF + ((b >> 16) & 1)) >> 16
    lo = r[:D // 2] & 0xFFFF
    hi = r[D // 2:] << 16
    xp = hi | lo
    z = jnp.concatenate(
        [xp[:, q * Q:(q + 1) * Q] for q in range(4)], axis=0)
    g_ref[...] = z.T


def _pack(wt):
    return pl.pallas_call(
        _pack_body,
        grid=(TGRID,),
        in_specs=[pl.BlockSpec((D, TCV), lambda i: (0, i))],
        out_specs=pl.BlockSpec((Q, 2 * D), lambda i: (i, 0)),
        out_shape=jax.ShapeDtypeStruct((TGRID * Q, 2 * D), jnp.int32),
    )(wt)


@functools.partial(
    pl.kernel,
    out_type=jax.ShapeDtypeStruct((B,), jnp.float32),
    mesh=_mesh,
    compiler_params=pltpu.CompilerParams(needs_layout_passes=False),
    scratch_types=[
        pltpu.VMEM((BPW,), jnp.int32),
        pltpu.VMEM((BPW,), jnp.int32),
        pltpu.VMEM((BPW,), jnp.int32),
        pltpu.VMEM((BPW,), jnp.int32),
        pltpu.VMEM((CH, 2 * D), jnp.int32),
        pltpu.VMEM((CH, 2 * D), jnp.int32),
        pltpu.VMEM((BPW,), jnp.float32),
        pltpu.SemaphoreType.DMA,
        pltpu.SemaphoreType.DMA,
    ],
)
def _sc_dots(target_hbm, context_hbm, g_in_hbm, g_out_hbm, out_hbm,
             idx_t, idx_c, slab_t, slab_c, buf_t, buf_c, dots,
             sem_t, sem_c):
    wid = lax.axis_index("s") * NC + lax.axis_index("c")
    base = wid * BPW

    pltpu.sync_copy(target_hbm.at[pl.ds(base, BPW)], idx_t)
    pltpu.sync_copy(context_hbm.at[pl.ds(base, BPW)], idx_c)

    def slab_body(g, carry):
        vt = idx_t[pl.ds(g * L, L)]
        vc = idx_c[pl.ds(g * L, L)]
        slab_t[pl.ds(g * L, L)] = ((vt >> 14) << 12) + (vt & 4095)
        slab_c[pl.ds(g * L, L)] = ((vc >> 14) << 12) + (vc & 4095)
        return carry

    lax.fori_loop(0, BPW // L, slab_body, 0)

    lanes = lax.iota(jnp.int32, L)

    for h in range(BPW // CH):
        cp_t = pltpu.async_copy(
            g_in_hbm.at[slab_t.at[pl.ds(h * CH, CH)]], buf_t, sem_t)
        cp_c = pltpu.async_copy(
            g_out_hbm.at[slab_c.at[pl.ds(h * CH, CH)]], buf_c, sem_c)
        cp_t.wait()
        cp_c.wait()

        def grp_body(gg, carry):
            s0 = h * CH + gg * L
            vt = idx_t[pl.ds(s0, L)]
            vc = idx_c[pl.ds(s0, L)]
            lid = gg * L + lanes
            ot = ((vt >> 12) & 3) * 32
            oc = ((vc >> 12) & 3) * 32
            acc = jnp.zeros((L,), jnp.float32)
            for dp in range(D // 2):
                tw = plsc.load_gather(buf_t, [lid, ot + dp])
                cw = plsc.load_gather(buf_c, [lid, oc + dp])
                tlo, thi = plsc.unpack(
                    plsc.bitcast(tw, jnp.bfloat16),
                    format=plsc.PackFormat.INTERLEAVED)
                clo, chi = plsc.unpack(
                    plsc.bitcast(cw, jnp.bfloat16),
                    format=plsc.PackFormat.INTERLEAVED)
                acc = acc + tlo * clo + thi * chi
            dots[pl.ds(s0, L)] = acc
            return carry

        lax.fori_loop(0, CH // L, grp_body, 0)

    pltpu.sync_copy(dots, out_hbm.at[pl.ds(base, BPW)])


def _bce_body(z_ref, y_ref, out_ref):
    z = z_ref[...]
    y = y_ref[...].astype(jnp.float32)
    p = jax.nn.sigmoid(z)
    eps = 1e-12
    p = jnp.clip(p, eps, 1.0 - eps)
    loss = y * jnp.log(p) + (1.0 - y) * jnp.log(1.0 - p)
    out_ref[0, 0] = -jnp.sum(loss) / B


def kernel(target, context, labels, W_in, W_out):
    g_in = _pack(W_in.T)
    g_out = _pack(W_out.T)
    dots = _sc_dots(target, context, g_in, g_out)
    loss = pl.pallas_call(
        _bce_body,
        out_shape=jax.ShapeDtypeStruct((1, 1), jnp.float32),
        out_specs=pl.BlockSpec(memory_space=pltpu.SMEM),
    )(dots.reshape(128, 128), labels.reshape(128, 128))
    return loss[0, 0]


# speedup vs baseline: 5.0587x; 1.0446x over previous
import functools

import jax
import jax.numpy as jnp
from jax import lax
from jax.experimental import pallas as pl
from jax.experimental.pallas import tpu as pltpu
from jax.experimental.pallas import tpu_sc as plsc

V = 1000000
D = 64
B = 16384
NC = 2
NS = 16
L = 16
NW = NC * NS
BPW = B // NW
CH = 256

TCV = 32768
Q = TCV // 4
TGRID = (V + TCV - 1) // TCV

_mesh = plsc.VectorSubcoreMesh(core_axis_name="c", subcore_axis_name="s")


def _pack_body(wt_ref, g_ref):
    x = wt_ref[...]
    b = lax.bitcast_convert_type(x, jnp.int32)
    r = (b + 0x7FFF + ((b >> 16) & 1)) >> 16
    lo = r[:D // 2] & 0xFFFF
    hi = r[D // 2:] << 16
    xp = hi | lo
    z = jnp.concatenate(
        [xp[:, q * Q:(q + 1) * Q] for q in range(4)], axis=0)
    g_ref[...] = z.T


def _pack(wt):
    return pl.pallas_call(
        _pack_body,
        grid=(TGRID,),
        in_specs=[pl.BlockSpec((D, TCV), lambda i: (0, i))],
        out_specs=pl.BlockSpec((Q, 2 * D), lambda i: (i, 0)),
        out_shape=jax.ShapeDtypeStruct((TGRID * Q, 2 * D), jnp.int32),
    )(wt)


@functools.partial(
    pl.kernel,
    out_type=jax.ShapeDtypeStruct((B,), jnp.float32),
    mesh=_mesh,
    compiler_params=pltpu.CompilerParams(needs_layout_passes=False),
    scratch_types=[
        pltpu.VMEM((BPW,), jnp.int32),
        pltpu.VMEM((BPW,), jnp.int32),
        pltpu.VMEM((BPW,), jnp.int32),
        pltpu.VMEM((BPW,), jnp.int32),
        pltpu.VMEM((CH, 2 * D), jnp.int32),
        pltpu.VMEM((CH, 2 * D), jnp.int32),
        pltpu.VMEM((BPW,), jnp.float32),
        pltpu.SemaphoreType.DMA,
        pltpu.SemaphoreType.DMA,
    ],
)
def _sc_dots(target_hbm, context_hbm, g_in_hbm, g_out_hbm, out_hbm,
             idx_t, idx_c, slab_t, slab_c, buf_t, buf_c, dots,
             sem_t, sem_c):
    wid = lax.axis_index("s") * NC + lax.axis_index("c")
    base = wid * BPW

    pltpu.sync_copy(target_hbm.at[pl.ds(base, BPW)], idx_t)
    pltpu.sync_copy(context_hbm.at[pl.ds(base, BPW)], idx_c)

    def slab_body(g, carry):
        vt = idx_t[pl.ds(g * L, L)]
        vc = idx_c[pl.ds(g * L, L)]
        slab_t[pl.ds(g * L, L)] = ((vt >> 15) << 13) + (vt & 8191)
        slab_c[pl.ds(g * L, L)] = ((vc >> 15) << 13) + (vc & 8191)
        return carry

    lax.fori_loop(0, BPW // L, slab_body, 0)

    lanes = lax.iota(jnp.int32, L)

    for h in range(BPW // CH):
        cp_t = pltpu.async_copy(
            g_in_hbm.at[slab_t.at[pl.ds(h * CH, CH)]], buf_t, sem_t)
        cp_c = pltpu.async_copy(
            g_out_hbm.at[slab_c.at[pl.ds(h * CH, CH)]], buf_c, sem_c)
        cp_t.wait()
        cp_c.wait()

        def grp_body(gg, carry):
            s0 = h * CH + gg * L
            vt = idx_t[pl.ds(s0, L)]
            vc = idx_c[pl.ds(s0, L)]
            lid = gg * L + lanes
            ot = ((vt >> 13) & 3) * 32
            oc = ((vc >> 13) & 3) * 32
            acc = jnp.zeros((L,), jnp.float32)
            for dp in range(D // 2):
                tw = plsc.load_gather(buf_t, [lid, ot + dp])
                cw = plsc.load_gather(buf_c, [lid, oc + dp])
                tlo, thi = plsc.unpack(
                    plsc.bitcast(tw, jnp.bfloat16),
                    format=plsc.PackFormat.INTERLEAVED)
                clo, chi = plsc.unpack(
                    plsc.bitcast(cw, jnp.bfloat16),
                    format=plsc.PackFormat.INTERLEAVED)
                acc = acc + tlo * clo + thi * chi
            dots[pl.ds(s0, L)] = acc
            return carry

        lax.fori_loop(0, CH // L, grp_body, 0)

    pltpu.sync_copy(dots, out_hbm.at[pl.ds(base, BPW)])


def _bce_body(z_ref, y_ref, out_ref):
    z = z_ref[...]
    y = y_ref[...].astype(jnp.float32)
    p = jax.nn.sigmoid(z)
    eps = 1e-12
    p = jnp.clip(p, eps, 1.0 - eps)
    loss = y * jnp.log(p) + (1.0 - y) * jnp.log(1.0 - p)
    out_ref[0, 0] = -jnp.sum(loss) / B


def kernel(target, context, labels, W_in, W_out):
    g_in = _pack(W_in.T)
    g_out = _pack(W_out.T)
    dots = _sc_dots(target, context, g_in, g_out)
    loss = pl.pallas_call(
        _bce_body,
        out_shape=jax.ShapeDtypeStruct((1, 1), jnp.float32),
        out_specs=pl.BlockSpec(memory_space=pltpu.SMEM),
    )(dots.reshape(128, 128), labels.reshape(128, 128))
    return loss[0, 0]


# speedup vs baseline: 5.1864x; 1.0252x over previous
import functools

import jax
import jax.numpy as jnp
from jax import lax
from jax.experimental import pallas as pl
from jax.experimental.pallas import tpu as pltpu
from jax.experimental.pallas import tpu_sc as plsc

V = 1000000
D = 64
B = 16384
NC = 2
NS = 16
L = 16
NW = NC * NS
BPW = B // NW
CH = 256

TCV = 32768
Q = TCV // 4
TGRID = (V + TCV - 1) // TCV

_mesh = plsc.VectorSubcoreMesh(core_axis_name="c", subcore_axis_name="s")


def _pack_body(wt_ref, g_ref):
    x = wt_ref[...]
    b = lax.bitcast_convert_type(x, jnp.int32)
    r = (b + 0x7FFF + ((b >> 16) & 1)) >> 16
    lo = r[:D // 2] & 0xFFFF
    hi = r[D // 2:] << 16
    xp = hi | lo
    z = jnp.concatenate(
        [xp[:, q * Q:(q + 1) * Q] for q in range(4)], axis=0)
    g_ref[...] = z.T


def _pack(wt):
    return pl.pallas_call(
        _pack_body,
        grid=(TGRID,),
        in_specs=[pl.BlockSpec((D, TCV), lambda i: (0, i))],
        out_specs=pl.BlockSpec((Q, 2 * D), lambda i: (i, 0)),
        out_shape=jax.ShapeDtypeStruct((TGRID * Q, 2 * D), jnp.int32),
    )(wt)


@functools.partial(
    pl.kernel,
    out_type=jax.ShapeDtypeStruct((NW, D // 2, BPW), jnp.int32),
    mesh=_mesh,
    compiler_params=pltpu.CompilerParams(needs_layout_passes=False),
    scratch_types=[
        pltpu.VMEM((BPW,), jnp.int32),
        pltpu.VMEM((BPW,), jnp.int32),
        pltpu.VMEM((CH, 2 * D), jnp.int32),
        pltpu.VMEM((D // 2, BPW), jnp.int32),
        pltpu.SemaphoreType.DMA,
    ],
)
def _sc_gather_t(target_hbm, g_in_hbm, out_hbm, idx_t, slab_t, buf_t, pt,
                 sem_t):
    wid = lax.axis_index("s") * NC + lax.axis_index("c")
    base = wid * BPW

    pltpu.sync_copy(target_hbm.at[pl.ds(base, BPW)], idx_t)

    def slab_body(g, carry):
        vt = idx_t[pl.ds(g * L, L)]
        slab_t[pl.ds(g * L, L)] = ((vt >> 15) << 13) + (vt & 8191)
        return carry

    lax.fori_loop(0, BPW // L, slab_body, 0)

    lanes = lax.iota(jnp.int32, L)

    for h in range(BPW // CH):
        pltpu.async_copy(
            g_in_hbm.at[slab_t.at[pl.ds(h * CH, CH)]], buf_t, sem_t).wait()

        def grp_body(gg, carry):
            s0 = h * CH + gg * L
            vt = idx_t[pl.ds(s0, L)]
            lid = gg * L + lanes
            ot = ((vt >> 13) & 3) * 32
            for dp in range(D // 2):
                pt[dp, pl.ds(s0, L)] = plsc.load_gather(buf_t, [lid, ot + dp])
            return carry

        lax.fori_loop(0, CH // L, grp_body, 0)

    pltpu.sync_copy(pt, out_hbm.at[wid])


@functools.partial(
    pl.kernel,
    out_type=jax.ShapeDtypeStruct((B,), jnp.float32),
    mesh=_mesh,
    compiler_params=pltpu.CompilerParams(needs_layout_passes=False),
    scratch_types=[
        pltpu.VMEM((BPW,), jnp.int32),
        pltpu.VMEM((BPW,), jnp.int32),
        pltpu.VMEM((CH, 2 * D), jnp.int32),
        pltpu.VMEM((D // 2, BPW), jnp.int32),
        pltpu.VMEM((BPW,), jnp.float32),
        pltpu.SemaphoreType.DMA,
    ],
)
def _sc_dots(context_hbm, g_out_hbm, pt_hbm, out_hbm,
             idx_c, slab_c, buf_c, pt, dots, sem_c):
    wid = lax.axis_index("s") * NC + lax.axis_index("c")
    base = wid * BPW

    pltpu.sync_copy(context_hbm.at[pl.ds(base, BPW)], idx_c)
    pltpu.sync_copy(pt_hbm.at[wid], pt)

    def slab_body(g, carry):
        vc = idx_c[pl.ds(g * L, L)]
        slab_c[pl.ds(g * L, L)] = ((vc >> 15) << 13) + (vc & 8191)
        return carry

    lax.fori_loop(0, BPW // L, slab_body, 0)

    lanes = lax.iota(jnp.int32, L)

    for h in range(BPW // CH):
        pltpu.async_copy(
            g_out_hbm.at[slab_c.at[pl.ds(h * CH, CH)]], buf_c, sem_c).wait()

        def grp_body(gg, carry):
            s0 = h * CH + gg * L
            vc = idx_c[pl.ds(s0, L)]
            lid = gg * L + lanes
            oc = ((vc >> 13) & 3) * 32
            acc = jnp.zeros((L,), jnp.float32)
            for dp in range(D // 2):
                tw = pt[dp, pl.ds(s0, L)]
                cw = plsc.load_gather(buf_c, [lid, oc + dp])
                tlo, thi = plsc.unpack(
                    plsc.bitcast(tw, jnp.bfloat16),
                    format=plsc.PackFormat.INTERLEAVED)
                clo, chi = plsc.unpack(
                    plsc.bitcast(cw, jnp.bfloat16),
                    format=plsc.PackFormat.INTERLEAVED)
                acc = acc + tlo * clo + thi * chi
            dots[pl.ds(s0, L)] = acc
            return carry

        lax.fori_loop(0, CH // L, grp_body, 0)

    pltpu.sync_copy(dots, out_hbm.at[pl.ds(base, BPW)])


def _bce_body(z_ref, y_ref, out_ref):
    z = z_ref[...]
    y = y_ref[...].astype(jnp.float32)
    p = jax.nn.sigmoid(z)
    eps = 1e-12
    p = jnp.clip(p, eps, 1.0 - eps)
    loss = y * jnp.log(p) + (1.0 - y) * jnp.log(1.0 - p)
    out_ref[0, 0] = -jnp.sum(loss) / B


def kernel(target, context, labels, W_in, W_out):
    g_in = _pack(W_in.T)
    pt = _sc_gather_t(target, g_in)
    g_out = _pack(W_out.T)
    dots = _sc_dots(context, g_out, pt)
    loss = pl.pallas_call(
        _bce_body,
        out_shape=jax.ShapeDtypeStruct((1, 1), jnp.float32),
        out_specs=pl.BlockSpec(memory_space=pltpu.SMEM),
    )(dots.reshape(128, 128), labels.reshape(128, 128))
    return loss[0, 0]


# speedup vs baseline: 5.2221x; 1.0069x over previous
import functools

import jax
import jax.numpy as jnp
from jax import lax
from jax.experimental import pallas as pl
from jax.experimental.pallas import tpu as pltpu
from jax.experimental.pallas import tpu_sc as plsc

V = 1000000
D = 64
B = 16384
NC = 2
NS = 16
L = 16
NW = NC * NS
BPW = B // NW
CH = 256

TCV = 65536
Q = TCV // 4
TGRID = (V + TCV - 1) // TCV

_mesh = plsc.VectorSubcoreMesh(core_axis_name="c", subcore_axis_name="s")


def _pack_body(wt_ref, g_ref):
    x = wt_ref[...]
    b = lax.bitcast_convert_type(x, jnp.int32)
    r = (b + 0x7FFF + ((b >> 16) & 1)) >> 16
    lo = r[:D // 2] & 0xFFFF
    hi = r[D // 2:] << 16
    xp = hi | lo
    z = jnp.concatenate(
        [xp[:, q * Q:(q + 1) * Q] for q in range(4)], axis=0)
    g_ref[...] = z.T


def _pack(wt):
    return pl.pallas_call(
        _pack_body,
        grid=(TGRID,),
        in_specs=[pl.BlockSpec((D, TCV), lambda i: (0, i))],
        out_specs=pl.BlockSpec((Q, 2 * D), lambda i: (i, 0)),
        out_shape=jax.ShapeDtypeStruct((TGRID * Q, 2 * D), jnp.int32),
    )(wt)


@functools.partial(
    pl.kernel,
    out_type=jax.ShapeDtypeStruct((NW, D // 2, BPW), jnp.int32),
    mesh=_mesh,
    compiler_params=pltpu.CompilerParams(needs_layout_passes=False),
    scratch_types=[
        pltpu.VMEM((BPW,), jnp.int32),
        pltpu.VMEM((BPW,), jnp.int32),
        pltpu.VMEM((CH, 2 * D), jnp.int32),
        pltpu.VMEM((D // 2, BPW), jnp.int32),
        pltpu.SemaphoreType.DMA,
    ],
)
def _sc_gather_t(target_hbm, g_in_hbm, out_hbm, idx_t, slab_t, buf_t, pt,
                 sem_t):
    wid = lax.axis_index("s") * NC + lax.axis_index("c")
    base = wid * BPW

    pltpu.sync_copy(target_hbm.at[pl.ds(base, BPW)], idx_t)

    def slab_body(g, carry):
        vt = idx_t[pl.ds(g * L, L)]
        slab_t[pl.ds(g * L, L)] = ((vt >> 16) << 14) + (vt & 16383)
        return carry

    lax.fori_loop(0, BPW // L, slab_body, 0)

    lanes = lax.iota(jnp.int32, L)

    for h in range(BPW // CH):
        pltpu.async_copy(
            g_in_hbm.at[slab_t.at[pl.ds(h * CH, CH)]], buf_t, sem_t).wait()

        def grp_body(gg, carry):
            s0 = h * CH + gg * L
            vt = idx_t[pl.ds(s0, L)]
            lid = gg * L + lanes
            ot = ((vt >> 14) & 3) * 32
            for dp in range(D // 2):
                pt[dp, pl.ds(s0, L)] = plsc.load_gather(buf_t, [lid, ot + dp])
            return carry

        lax.fori_loop(0, CH // L, grp_body, 0)

    pltpu.sync_copy(pt, out_hbm.at[wid])


@functools.partial(
    pl.kernel,
    out_type=jax.ShapeDtypeStruct((B,), jnp.float32),
    mesh=_mesh,
    compiler_params=pltpu.CompilerParams(needs_layout_passes=False),
    scratch_types=[
        pltpu.VMEM((BPW,), jnp.int32),
        pltpu.VMEM((BPW,), jnp.int32),
        pltpu.VMEM((CH, 2 * D), jnp.int32),
        pltpu.VMEM((D // 2, BPW), jnp.int32),
        pltpu.VMEM((BPW,), jnp.float32),
        pltpu.SemaphoreType.DMA,
    ],
)
def _sc_dots(context_hbm, g_out_hbm, pt_hbm, out_hbm,
             idx_c, slab_c, buf_c, pt, dots, sem_c):
    wid = lax.axis_index("s") * NC + lax.axis_index("c")
    base = wid * BPW

    pltpu.sync_copy(context_hbm.at[pl.ds(base, BPW)], idx_c)
    pltpu.sync_copy(pt_hbm.at[wid], pt)

    def slab_body(g, carry):
        vc = idx_c[pl.ds(g * L, L)]
        slab_c[pl.ds(g * L, L)] = ((vc >> 16) << 14) + (vc & 16383)
        return carry

    lax.fori_loop(0, BPW // L, slab_body, 0)

    lanes = lax.iota(jnp.int32, L)

    for h in range(BPW // CH):
        pltpu.async_copy(
            g_out_hbm.at[slab_c.at[pl.ds(h * CH, CH)]], buf_c, sem_c).wait()

        def grp_body(gg, carry):
            s0 = h * CH + gg * L
            vc = idx_c[pl.ds(s0, L)]
            lid = gg * L + lanes
            oc = ((vc >> 14) & 3) * 32
            acc = jnp.zeros((L,), jnp.float32)
            for dp in range(D // 2):
                tw = pt[dp, pl.ds(s0, L)]
                cw = plsc.load_gather(buf_c, [lid, oc + dp])
                tlo, thi = plsc.unpack(
                    plsc.bitcast(tw, jnp.bfloat16),
                    format=plsc.PackFormat.INTERLEAVED)
                clo, chi = plsc.unpack(
                    plsc.bitcast(cw, jnp.bfloat16),
                    format=plsc.PackFormat.INTERLEAVED)
                acc = acc + tlo * clo + thi * chi
            dots[pl.ds(s0, L)] = acc
            return carry

        lax.fori_loop(0, CH // L, grp_body, 0)

    pltpu.sync_copy(dots, out_hbm.at[pl.ds(base, BPW)])


def _bce_body(z_ref, y_ref, out_ref):
    z = z_ref[...]
    y = y_ref[...].astype(jnp.float32)
    p = jax.nn.sigmoid(z)
    eps = 1e-12
    p = jnp.clip(p, eps, 1.0 - eps)
    loss = y * jnp.log(p) + (1.0 - y) * jnp.log(1.0 - p)
    out_ref[0, 0] = -jnp.sum(loss) / B


def kernel(target, context, labels, W_in, W_out):
    g_in = _pack(W_in.T)
    pt = _sc_gather_t(target, g_in)
    g_out = _pack(W_out.T)
    dots = _sc_dots(context, g_out, pt)
    loss = pl.pallas_call(
        _bce_body,
        out_shape=jax.ShapeDtypeStruct((1, 1), jnp.float32),
        out_specs=pl.BlockSpec(memory_space=pltpu.SMEM),
    )(dots.reshape(128, 128), labels.reshape(128, 128))
    return loss[0, 0]


# speedup vs baseline: 5.2660x; 1.0084x over previous
import functools

import jax
import jax.numpy as jnp
from jax import lax
from jax.experimental import pallas as pl
from jax.experimental.pallas import tpu as pltpu
from jax.experimental.pallas import tpu_sc as plsc

V = 1000000
D = 64
B = 16384
NC = 2
NS = 16
L = 16
NW = NC * NS
BPW = B // NW
CH = 512

TCV = 65536
Q = TCV // 4
TGRID = (V + TCV - 1) // TCV

_mesh = plsc.VectorSubcoreMesh(core_axis_name="c", subcore_axis_name="s")


def _pack_body(wt_ref, g_ref):
    x = wt_ref[...]
    b = lax.bitcast_convert_type(x, jnp.int32)
    r = (b + 0x7FFF + ((b >> 16) & 1)) >> 16
    lo = r[:D // 2] & 0xFFFF
    hi = r[D // 2:] << 16
    xp = hi | lo
    z = jnp.concatenate(
        [xp[:, q * Q:(q + 1) * Q] for q in range(4)], axis=0)
    g_ref[...] = z.T


def _pack(wt):
    return pl.pallas_call(
        _pack_body,
        grid=(TGRID,),
        in_specs=[pl.BlockSpec((D, TCV), lambda i: (0, i))],
        out_specs=pl.BlockSpec((Q, 2 * D), lambda i: (i, 0)),
        out_shape=jax.ShapeDtypeStruct((TGRID * Q, 2 * D), jnp.int32),
    )(wt)


@functools.partial(
    pl.kernel,
    out_type=jax.ShapeDtypeStruct((NW, D // 2, BPW), jnp.int32),
    mesh=_mesh,
    compiler_params=pltpu.CompilerParams(needs_layout_passes=False),
    scratch_types=[
        pltpu.VMEM((BPW,), jnp.int32),
        pltpu.VMEM((BPW,), jnp.int32),
        pltpu.VMEM((CH, 2 * D), jnp.int32),
        pltpu.VMEM((D // 2, BPW), jnp.int32),
        pltpu.SemaphoreType.DMA,
    ],
)
def _sc_gather_t(target_hbm, g_in_hbm, out_hbm, idx_t, slab_t, buf_t, pt,
                 sem_t):
    wid = lax.axis_index("s") * NC + lax.axis_index("c")
    base = wid * BPW

    pltpu.sync_copy(target_hbm.at[pl.ds(base, BPW)], idx_t)

    def slab_body(g, carry):
        vt = idx_t[pl.ds(g * L, L)]
        slab_t[pl.ds(g * L, L)] = ((vt >> 16) << 14) + (vt & 16383)
        return carry

    lax.fori_loop(0, BPW // L, slab_body, 0)

    lanes = lax.iota(jnp.int32, L)

    for h in range(BPW // CH):
        pltpu.async_copy(
            g_in_hbm.at[slab_t.at[pl.ds(h * CH, CH)]], buf_t, sem_t).wait()

        def grp_body(gg, carry):
            s0 = h * CH + gg * L
            vt = idx_t[pl.ds(s0, L)]
            lid = gg * L + lanes
            ot = ((vt >> 14) & 3) * 32
            for dp in range(D // 2):
                pt[dp, pl.ds(s0, L)] = plsc.load_gather(buf_t, [lid, ot + dp])
            return carry

        lax.fori_loop(0, CH // L, grp_body, 0)

    pltpu.sync_copy(pt, out_hbm.at[wid])


@functools.partial(
    pl.kernel,
    out_type=jax.ShapeDtypeStruct((B,), jnp.float32),
    mesh=_mesh,
    compiler_params=pltpu.CompilerParams(needs_layout_passes=False),
    scratch_types=[
        pltpu.VMEM((BPW,), jnp.int32),
        pltpu.VMEM((BPW,), jnp.int32),
        pltpu.VMEM((CH, 2 * D), jnp.int32),
        pltpu.VMEM((D // 2, BPW), jnp.int32),
        pltpu.VMEM((BPW,), jnp.float32),
        pltpu.SemaphoreType.DMA,
    ],
)
def _sc_dots(context_hbm, g_out_hbm, pt_hbm, out_hbm,
             idx_c, slab_c, buf_c, pt, dots, sem_c):
    wid = lax.axis_index("s") * NC + lax.axis_index("c")
    base = wid * BPW

    pltpu.sync_copy(context_hbm.at[pl.ds(base, BPW)], idx_c)
    pltpu.sync_copy(pt_hbm.at[wid], pt)

    def slab_body(g, carry):
        vc = idx_c[pl.ds(g * L, L)]
        slab_c[pl.ds(g * L, L)] = ((vc >> 16) << 14) + (vc & 16383)
        return carry

    lax.fori_loop(0, BPW // L, slab_body, 0)

    lanes = lax.iota(jnp.int32, L)

    for h in range(BPW // CH):
        pltpu.async_copy(
            g_out_hbm.at[slab_c.at[pl.ds(h * CH, CH)]], buf_c, sem_c).wait()

        def grp_body(gg, carry):
            s0 = h * CH + gg * L
            vc = idx_c[pl.ds(s0, L)]
            lid = gg * L + lanes
            oc = ((vc >> 14) & 3) * 32
            acc = jnp.zeros((L,), jnp.float32)
            for dp in range(D // 2):
                tw = pt[dp, pl.ds(s0, L)]
                cw = plsc.load_gather(buf_c, [lid, oc + dp])
                tlo, thi = plsc.unpack(
                    plsc.bitcast(tw, jnp.bfloat16),
                    format=plsc.PackFormat.INTERLEAVED)
                clo, chi = plsc.unpack(
                    plsc.bitcast(cw, jnp.bfloat16),
                    format=plsc.PackFormat.INTERLEAVED)
                acc = acc + tlo * clo + thi * chi
            dots[pl.ds(s0, L)] = acc
            return carry

        lax.fori_loop(0, CH // L, grp_body, 0)

    pltpu.sync_copy(dots, out_hbm.at[pl.ds(base, BPW)])


def _bce_body(z_ref, y_ref, out_ref):
    z = z_ref[...]
    y = y_ref[...].astype(jnp.float32)
    p = jax.nn.sigmoid(z)
    eps = 1e-12
    p = jnp.clip(p, eps, 1.0 - eps)
    loss = y * jnp.log(p) + (1.0 - y) * jnp.log(1.0 - p)
    out_ref[0, 0] = -jnp.sum(loss) / B


def kernel(target, context, labels, W_in, W_out):
    g_in = _pack(W_in.T)
    pt = _sc_gather_t(target, g_in)
    g_out = _pack(W_out.T)
    dots = _sc_dots(context, g_out, pt)
    loss = pl.pallas_call(
        _bce_body,
        out_shape=jax.ShapeDtypeStruct((1, 1), jnp.float32),
        out_specs=pl.BlockSpec(memory_space=pltpu.SMEM),
    )(dots.reshape(128, 128), labels.reshape(128, 128))
    return loss[0, 0]
